# Initial kernel scaffold; baseline (speedup 1.0000x reference)
import jax
import jax.numpy as jnp
import numpy as np
from jax import lax
from jax.experimental import pallas as pl
from jax.experimental.pallas import tpu as pltpu
from jax.experimental.pallas import tpu_sc as plsc

_WDX = 1.0
_WLB = 1.0
_WC = 1.0
_YLEN = 10
_RC = [12, 5, 12.36, 18, 9.1, 10, 18.62, 1.5, 18, 1.5]
_COSTS16 = np.zeros(16, np.float32)
_COSTS16[:_YLEN] = np.array(_RC, np.float64) / np.sum(_RC)

_B = 128
_T = 512
_F = 512
_NW = 32
_SPW = _B // _NW
_CSLICE = 32


def _body(xprev, xpost, tcat, costs, out, bufp, bufq, tbuf, cbuf, vf, orow):
    nc = 2
    wid = lax.axis_index("s") * nc + lax.axis_index("c")
    base = wid * _SPW

    pltpu.sync_copy(costs, cbuf)
    costv = cbuf[...]
    iot = lax.iota(jnp.int32, 16)
    half = jnp.full((16,), 0.5, jnp.float32)
    kslope = jnp.full((16,), 1000.0, jnp.float32)
    one = jnp.full((16,), 1.0, jnp.float32)
    zero = jnp.zeros((16,), jnp.float32)
    invT = jnp.float32(1.0 / _T)

    vf[pl.ds(32, 16)] = zero

    orow_acc = zero
    for j in range(_SPW):
        b = base + j
        pltpu.sync_copy(xprev.at[b, :, pl.ds(0, _CSLICE)], bufp)
        pltpu.sync_copy(xpost.at[b, :, pl.ds(0, _CSLICE)], bufq)
        pltpu.sync_copy(tcat.at[b], tbuf)

        def tstep(tt, carry):
            a0, a1, c0, c1 = carry
            a0 = a0 + bufp[tt, pl.ds(0, 16)]
            a1 = a1 + bufp[tt, pl.ds(16, 16)]
            c0 = c0 + bufq[tt, pl.ds(0, 16)]
            c1 = c1 + bufq[tt, pl.ds(16, 16)]
            return a0, a1, c0, c1

        a0, a1, c0, c1 = lax.fori_loop(0, _T, tstep, (zero, zero, zero, zero))
        vf[pl.ds(0, 16)] = jnp.abs(a0 - c0) * invT
        vf[pl.ds(16, 16)] = jnp.abs(a1 - c1) * invT

        g0 = plsc.load_gather(vf, [iot * 3])
        g1 = plsc.load_gather(vf, [iot * 3 + 1])
        g2 = plsc.load_gather(vf, [iot * 3 + 2])
        varm_lab = g0 + g1 + g2

        tv = tbuf[pl.ds(0, 16)]
        tupv = tbuf[pl.ds(16, 16)]
        tlowv = tbuf[pl.ds(32, 16)]

        thisorder = one / (one + jnp.exp((half - tv) * kslope))
        order_mask = jnp.where(tv > half, one, zero)
        eachval = jnp.sum(order_mask * varm_lab * thisorder)
        bmask = jnp.where(tupv == tlowv, one, zero)
        eachb = jnp.sum(bmask * jnp.abs(tv - tlowv))
        cost = jnp.sum(thisorder * costv)
        labvar = eachval * _WDX - eachb * _WLB - cost * _WC

        orow_acc = jnp.where(iot == j, labvar, orow_acc)
        orow_acc = jnp.where(iot == 4 + j, eachval, orow_acc)
        orow_acc = jnp.where(iot == 8 + j, eachb, orow_acc)
        orow_acc = jnp.where(iot == 12 + j, cost, orow_acc)

    orow[...] = orow_acc
    pltpu.sync_copy(orow, out.at[wid])


@jax.jit
def kernel(Xprev, Xpost, t, tup, tlow):
    tcat = jnp.concatenate(
        [
            jnp.pad(t, ((0, 0), (0, 6))),
            jnp.pad(tup, ((0, 0), (0, 6))),
            jnp.pad(tlow, ((0, 0), (0, 6))),
        ],
        axis=1,
    )
    costs = jnp.asarray(_COSTS16)

    mesh = plsc.VectorSubcoreMesh(core_axis_name="c", subcore_axis_name="s")
    run = pl.kernel(
        _body,
        out_type=jax.ShapeDtypeStruct((_NW, 16), jnp.float32),
        mesh=mesh,
        scratch_types=[
            pltpu.VMEM((_T, _CSLICE), jnp.float32),
            pltpu.VMEM((_T, _CSLICE), jnp.float32),
            pltpu.VMEM((48,), jnp.float32),
            pltpu.VMEM((16,), jnp.float32),
            pltpu.VMEM((48,), jnp.float32),
            pltpu.VMEM((16,), jnp.float32),
        ],
    )
    res = run(Xprev, Xpost, tcat, costs)
    labvar = res[:, 0:4].reshape(_B)
    eachval_mean = jnp.mean(res[:, 4:8])
    eachb_mean = jnp.mean(res[:, 8:12])
    cost_mean = jnp.mean(res[:, 12:16])
    return (labvar, eachval_mean, eachb_mean, cost_mean)


# speedup vs baseline: 3.1164x; 3.1164x over previous
import jax
import jax.numpy as jnp
import numpy as np
from jax import lax
from jax.experimental import pallas as pl
from jax.experimental.pallas import tpu as pltpu
from jax.experimental.pallas import tpu_sc as plsc

_WDX = 1.0
_WLB = 1.0
_WC = 1.0
_YLEN = 10
_RC = [12, 5, 12.36, 18, 9.1, 10, 18.62, 1.5, 18, 1.5]
_COSTS16 = np.zeros(16, np.float32)
_COSTS16[:_YLEN] = np.array(_RC, np.float64) / np.sum(_RC)

_B = 128
_T = 512
_F = 512
_NW = 32
_SPW = _B // _NW
_CSLICE = 128
_TCH = 256
_NCH = _T // _TCH


def _sigmoid_shifted(tv, half, kslope, one):
    return one / (one + jnp.exp((half - tv) * kslope))


def _body(xprev, xpost, tcat, out, buf, tvm, orow):
    nc = 2
    wid = lax.axis_index("s") * nc + lax.axis_index("c")
    base = wid * _SPW

    pltpu.sync_copy(tcat, tvm)

    iot = lax.iota(jnp.int32, 16)
    half = jnp.full((16,), 0.5, jnp.float32)
    kslope = jnp.full((16,), 1000.0, jnp.float32)
    one = jnp.full((16,), 1.0, jnp.float32)
    zero = jnp.zeros((16,), jnp.float32)
    invT = jnp.float32(1.0 / _T)

    orow_acc = zero
    for j in range(_SPW):
        b = base + j

        def reduce_one(src):
            a0 = zero
            a1 = zero
            for k in range(_NCH):
                pltpu.sync_copy(
                    src.at[b, pl.ds(k * _TCH, _TCH), pl.ds(0, _CSLICE)], buf
                )

                def tstep(tt, carry):
                    c0, c1 = carry
                    c0 = c0 + buf[tt, pl.ds(0, 16)]
                    c1 = c1 + buf[tt, pl.ds(16, 16)]
                    return c0, c1

                a0, a1 = lax.fori_loop(0, _TCH, tstep, (a0, a1))
            return a0, a1

        p0, p1 = reduce_one(xprev)
        q0, q1 = reduce_one(xpost)
        vm0 = jnp.abs(p0 - q0) * invT
        vm1 = jnp.abs(p1 - q1) * invT

        te0 = tvm[b, pl.ds(0, 16)]
        te1 = tvm[b, pl.ds(16, 16)]
        tv = tvm[b, pl.ds(32, 16)]
        tupv = tvm[b, pl.ds(48, 16)]
        tlowv = tvm[b, pl.ds(64, 16)]
        costv = tvm[b, pl.ds(80, 16)]

        w0 = jnp.where(te0 > half, one, zero) * _sigmoid_shifted(te0, half, kslope, one)
        w1 = jnp.where(te1 > half, one, zero) * _sigmoid_shifted(te1, half, kslope, one)
        eachval = jnp.sum(vm0 * w0) + jnp.sum(vm1 * w1)

        bmask = jnp.where(tupv == tlowv, one, zero)
        eachb = jnp.sum(bmask * jnp.abs(tv - tlowv))
        thisorder = _sigmoid_shifted(tv, half, kslope, one)
        cost = jnp.sum(thisorder * costv)
        labvar = eachval * _WDX - eachb * _WLB - cost * _WC

        orow_acc = jnp.where(iot == j, labvar, orow_acc)
        orow_acc = jnp.where(iot == 4 + j, eachval, orow_acc)
        orow_acc = jnp.where(iot == 8 + j, eachb, orow_acc)
        orow_acc = jnp.where(iot == 12 + j, cost, orow_acc)

    orow[pl.ds(0, 16)] = orow_acc
    for k in range(1, 8):
        orow[pl.ds(k * 16, 16)] = zero
    pltpu.sync_copy(orow, out.at[pl.ds(wid * 128, 128)])


@jax.jit
def kernel(Xprev, Xpost, t, tup, tlow):
    texp = jnp.pad(jnp.repeat(t, 3, axis=1), ((0, 0), (0, 2)))
    costs_row = jnp.broadcast_to(jnp.asarray(_COSTS16), (_B, 16))
    tcat = jnp.concatenate(
        [
            texp,
            jnp.pad(t, ((0, 0), (0, 6))),
            jnp.pad(tup, ((0, 0), (0, 6))),
            jnp.pad(tlow, ((0, 0), (0, 6))),
            costs_row,
            jnp.zeros((_B, 32), jnp.float32),
        ],
        axis=1,
    )

    mesh = plsc.VectorSubcoreMesh(
        core_axis_name="c", subcore_axis_name="s", num_cores=2, num_subcores=16
    )
    run = pl.kernel(
        _body,
        out_type=jax.ShapeDtypeStruct((_NW * 128,), jnp.float32),
        mesh=mesh,
        scratch_types=[
            pltpu.VMEM((_TCH, _CSLICE), jnp.float32),
            pltpu.VMEM((_B, 128), jnp.float32),
            pltpu.VMEM((128,), jnp.float32),
        ],
        compiler_params=pltpu.CompilerParams(needs_layout_passes=False),
    )
    res = run(Xprev, Xpost, tcat).reshape(_NW, 128)
    labvar = res[:, 0:4].reshape(_B)
    eachval_mean = jnp.mean(res[:, 4:8])
    eachb_mean = jnp.mean(res[:, 8:12])
    cost_mean = jnp.mean(res[:, 12:16])
    return (labvar, eachval_mean, eachb_mean, cost_mean)
